# Initial kernel scaffold; baseline (speedup 1.0000x reference)
import jax
import jax.numpy as jnp
from jax.experimental import pallas as pl


def kernel(tok, pe_weight):
    raise NotImplementedError("write your pallas kernel here")


# speedup vs baseline: 6.1245x; 6.1245x over previous
import functools

import jax
import jax.numpy as jnp
from jax import lax
from jax.experimental import pallas as pl
from jax.experimental.pallas import tpu as pltpu
from jax.experimental.pallas import tpu_sc as plsc

_INFO = plsc.get_sparse_core_info()
_NC = _INFO.num_cores
_NS = _INFO.num_subcores
_NW = _NC * _NS


@functools.partial(jax.jit, static_argnames=("chunk",))
def _sc_gather(tok_flat, pe_weight, chunk=1024):
    B, = tok_flat.shape
    V, D = pe_weight.shape
    b_per_w = B // _NW
    n_chunks = b_per_w // chunk
    assert b_per_w % chunk == 0 and B % _NW == 0

    mesh = plsc.VectorSubcoreMesh(core_axis_name="c", subcore_axis_name="s")

    @functools.partial(
        pl.kernel,
        mesh=mesh,
        out_type=jax.ShapeDtypeStruct((B, D), jnp.float32),
        scratch_types=[
            pltpu.VMEM((chunk,), jnp.int32),
            pltpu.VMEM((chunk, D), jnp.float32),
            pltpu.SemaphoreType.DMA,
        ],
        compiler_params=pltpu.CompilerParams(use_tc_tiling_on_sc=False),
    )
    def k(idx_hbm, table_hbm, out_hbm, idx_v, rows_v, sem):
        wid = lax.axis_index("s") * _NC + lax.axis_index("c")
        base = wid * b_per_w

        def body(g, carry):
            off = base + g * chunk
            pltpu.sync_copy(idx_hbm.at[pl.ds(off, chunk)], idx_v)
            pltpu.async_copy(table_hbm.at[idx_v], rows_v, sem).wait()
            pltpu.sync_copy(rows_v, out_hbm.at[pl.ds(off, chunk)])
            return carry

        lax.fori_loop(0, n_chunks, body, 0)

    return k(tok_flat, pe_weight)


def kernel(tok, pe_weight):
    B0, S = tok.shape
    V, D = pe_weight.shape
    out = _sc_gather(tok.reshape(B0 * S), pe_weight)
    return out.reshape(B0, S, D)


# speedup vs baseline: 6.2737x; 1.0244x over previous
import functools

import jax
import jax.numpy as jnp
from jax import lax
from jax.experimental import pallas as pl
from jax.experimental.pallas import tpu as pltpu
from jax.experimental.pallas import tpu_sc as plsc

_INFO = plsc.get_sparse_core_info()
_NC = _INFO.num_cores
_NS = _INFO.num_subcores
_NW = _NC * _NS


@functools.partial(jax.jit, static_argnames=("rows_per_chunk",))
def _sc_gather(tok, pe_weight, rows_per_chunk=8):
    B0, S = tok.shape
    V, D = pe_weight.shape
    R = rows_per_chunk
    rows_per_w = B0 // _NW
    n_chunks = rows_per_w // R
    assert B0 % _NW == 0 and rows_per_w % R == 0

    mesh = plsc.VectorSubcoreMesh(core_axis_name="c", subcore_axis_name="s")

    @functools.partial(
        pl.kernel,
        mesh=mesh,
        out_type=jax.ShapeDtypeStruct((B0, S, D), jnp.float32),
        scratch_types=[
            pltpu.VMEM((R, S), jnp.int32),
            pltpu.VMEM((R, S, D), jnp.float32),
            pltpu.SemaphoreType.DMA,
        ],
        compiler_params=pltpu.CompilerParams(use_tc_tiling_on_sc=False),
    )
    def k(idx_hbm, table_hbm, out_hbm, idx_v, rows_v, sem):
        wid = lax.axis_index("s") * _NC + lax.axis_index("c")
        base = wid * rows_per_w

        def body(g, carry):
            row = base + g * R
            pltpu.sync_copy(idx_hbm.at[pl.ds(row, R)], idx_v)
            copies = [
                pltpu.async_copy(table_hbm.at[idx_v.at[r]], rows_v.at[r], sem)
                for r in range(R)
            ]
            for c in copies:
                c.wait()
            pltpu.sync_copy(rows_v, out_hbm.at[pl.ds(row, R)])
            return carry

        lax.fori_loop(0, n_chunks, body, 0)

    return k(tok, pe_weight)


def kernel(tok, pe_weight):
    return _sc_gather(tok, pe_weight)
